# baseline (speedup 1.0000x reference)
import functools

import jax
import jax.numpy as jnp
from jax import lax
from jax.experimental import pallas as pl
from jax.experimental.pallas import tpu as pltpu
from jax.experimental.pallas import tpu_sc as plsc

B = 16384
UNIFIED = 64
DIM_MID = 32
DIM_TAIL = 16
NC = 2
NS = 16
NW = NC * NS
RPW = B // NW
GCH = 128
LSZ = RPW + 144


def _body(x_hbm, g_hbm, head_hbm, mid_hbm, tail_hbm, wtm_hbm, wtt_hbm,
          bm_hbm, bt_hbm, out_hbm,
          xv, gv, idxL, posL, buf, outb, wtmv, wttv, bmv, btv,
          sem):
    wid = lax.axis_index("s") * NC + lax.axis_index("c")

    pltpu.sync_copy(x_hbm.at[wid], xv.at[pl.ds(0, RPW)])
    pltpu.sync_copy(g_hbm.at[wid], gv.at[pl.ds(0, RPW)])
    pltpu.sync_copy(wtm_hbm, wtmv)
    pltpu.sync_copy(wtt_hbm, wttv)
    pltpu.sync_copy(bm_hbm, bmv)
    pltpu.sync_copy(bt_hbm, btv)

    zi = jnp.zeros((16,), jnp.int32)
    for t in range(3 * LSZ // 16):
        idxL[pl.ds(t * 16, 16)] = zi

    def crow(r, offs):
        o0, o1, o2 = offs
        xval = xv[pl.ds(r, 16)][0]
        g = gv[pl.ds(r, 16)][0]
        m0 = ((2 - g) * (1 - g)) >> 1
        m1 = g * (2 - g)
        m2 = (g * (g - 1)) >> 1
        p = o0 * m0 + (LSZ + o1) * m1 + (2 * LSZ + o2) * m2
        xs = (xval >> 1) * m0 + (xval >> 2) * m1 + (xval >> 3) * m2
        rem = (xval & 1) * m0 + (xval & 3) * m1 + (xval & 7) * m2
        idxL[pl.ds(p, 16)] = xs + zi
        posL[pl.ds(p, 16)] = (r | (rem << 9)) + zi
        return (o0 + m0, o1 + m1, o2 + m2)

    n0, n1, n2 = lax.fori_loop(
        0, RPW, crow, (jnp.int32(0), jnp.int32(0), jnp.int32(0)))

    zf = jnp.zeros((16,), jnp.float32)

    def process_group(tbl_hbm, base, n, row_fn):
        def chunk(jc, c):
            cp = pltpu.async_copy(
                tbl_hbm.at[idxL.at[pl.ds(base + jc * GCH, GCH)]],
                buf, sem)
            cp.wait()
            rem = n - jc * GCH
            t = rem - GCH
            nrem = GCH + (t & (t >> 31))

            def row(i, cc):
                pr = posL[pl.ds(base + jc * GCH + i, 16)][0]
                row_fn(i, pr & 511, pr >> 9)
                return cc
            lax.fori_loop(0, nrem, row, 0)
            return c
        lax.fori_loop(0, (n + GCH - 1) // GCH, chunk, 0)

    def hrow(i, pos, rem):
        cb = rem * UNIFIED
        for j in range(4):
            outb[pos, pl.ds(j * 16, 16)] = buf[i, pl.ds(cb + j * 16, 16)]
    process_group(head_hbm, 0, n0, hrow)

    def mrow(i, pos, rem):
        cb = rem * DIM_MID
        ev = [buf[i, pl.ds(cb + h * 16, 16)] for h in range(2)]
        acc = [bmv[pl.ds(j * 16, 16)] for j in range(4)]
        ac2 = [zf, zf, zf, zf]
        for k in range(DIM_MID):
            e = ev[k // 16][k % 16]
            tgt = acc if (k & 1) == 0 else ac2
            for j in range(4):
                tgt[j] = tgt[j] + e * wtmv[k, pl.ds(j * 16, 16)]
        for j in range(4):
            outb[pos, pl.ds(j * 16, 16)] = acc[j] + ac2[j]
    process_group(mid_hbm, LSZ, n1, mrow)

    def trow(i, pos, rem):
        cb = rem * DIM_TAIL
        ev = buf[i, pl.ds(cb, 16)]
        acc = [btv[pl.ds(j * 16, 16)] for j in range(4)]
        ac2 = [zf, zf, zf, zf]
        for k in range(DIM_TAIL):
            e = ev[k]
            tgt = acc if (k & 1) == 0 else ac2
            for j in range(4):
                tgt[j] = tgt[j] + e * wttv[k, pl.ds(j * 16, 16)]
        for j in range(4):
            outb[pos, pl.ds(j * 16, 16)] = acc[j] + ac2[j]
    process_group(tail_hbm, 2 * LSZ, n2, trow)

    pltpu.sync_copy(outb, out_hbm.at[wid])


@jax.jit
def _run(xr, gr, head_table, mid_table, tail_table, wtm, wtt, b_mid, b_tail):
    mesh = plsc.VectorSubcoreMesh(core_axis_name="c", subcore_axis_name="s")
    f = functools.partial(
        pl.kernel,
        mesh=mesh,
        out_type=jax.ShapeDtypeStruct((NW, RPW, UNIFIED), jnp.float32),
        scratch_types=[
            pltpu.VMEM((RPW + 16,), jnp.int32),
            pltpu.VMEM((RPW + 16,), jnp.int32),
            pltpu.VMEM((3 * LSZ,), jnp.int32),
            pltpu.VMEM((3 * LSZ,), jnp.int32),
            pltpu.VMEM((GCH, 128), jnp.float32),
            pltpu.VMEM((RPW, UNIFIED), jnp.float32),
            pltpu.VMEM((DIM_MID, UNIFIED), jnp.float32),
            pltpu.VMEM((DIM_TAIL, UNIFIED), jnp.float32),
            pltpu.VMEM((UNIFIED,), jnp.float32),
            pltpu.VMEM((UNIFIED,), jnp.float32),
            pltpu.SemaphoreType.DMA,
        ],
    )(_body)
    return f(xr, gr, head_table, mid_table, tail_table, wtm, wtt, b_mid,
             b_tail)


def kernel(x, frequency_groups, head_table, mid_table, tail_table, W_mid,
           b_mid, W_tail, b_tail):
    xr = x.reshape(NW, RPW)
    gr = frequency_groups.reshape(NW, RPW)
    wtm = W_mid.T
    wtt = W_tail.T
    head2 = head_table.reshape(-1, 128)
    mid2 = mid_table.reshape(-1, 128)
    tail2 = tail_table.reshape(-1, 128)
    out = _run(xr, gr, head2, mid2, tail2, wtm, wtt, b_mid, b_tail)
    return out.reshape(B, UNIFIED)


# speedup vs baseline: 1.2256x; 1.2256x over previous
import functools

import jax
import jax.numpy as jnp
from jax import lax
from jax.experimental import pallas as pl
from jax.experimental.pallas import tpu as pltpu
from jax.experimental.pallas import tpu_sc as plsc

B = 16384
UNIFIED = 64
DIM_MID = 32
DIM_TAIL = 16
NC = 2
NS = 16
NW = NC * NS
RPW = B // NW
GCH = 128
NCH = RPW // GCH


def _body(xr_hbm, gr_hbm, head_hbm, mid_hbm, tail_hbm, wtm_hbm, bm_hbm,
          wtt_hbm, bt_hbm, out_hbm,
          xidx, gv, bufh, bufm, buft, outb, wtmv, bmv, wttv, btv, sem):
    wid = lax.axis_index("s") * NC + lax.axis_index("c")

    pltpu.sync_copy(xr_hbm.at[wid], xidx)
    pltpu.sync_copy(gr_hbm.at[wid], gv.at[pl.ds(0, RPW)])
    pltpu.sync_copy(wtm_hbm, wtmv)
    pltpu.sync_copy(bm_hbm, bmv)
    pltpu.sync_copy(wtt_hbm, wttv)
    pltpu.sync_copy(bt_hbm, btv)

    cps = []
    for j in range(NCH):
        cps.append(pltpu.async_copy(
            head_hbm.at[xidx.at[j]], bufh.at[pl.ds(j * GCH, GCH)], sem))
        cps.append(pltpu.async_copy(
            mid_hbm.at[xidx.at[j]], bufm.at[pl.ds(j * GCH, GCH)], sem))
        cps.append(pltpu.async_copy(
            tail_hbm.at[xidx.at[j]], buft.at[pl.ds(j * GCH, GCH)], sem))
    for c in cps:
        c.wait()

    def row(r, carry):
        g = gv[pl.ds(r, 16)][0]

        @pl.when(g == 0)
        def _():
            for j in range(4):
                outb[r, pl.ds(j * 16, 16)] = bufh[r, pl.ds(j * 16, 16)]

        @pl.when(g == 1)
        def _():
            ev = [bufm[r, pl.ds(h * 16, 16)] for h in range(DIM_MID // 16)]
            acc = [bmv[pl.ds(j * 16, 16)] for j in range(4)]
            for k in range(DIM_MID):
                e = ev[k // 16][k % 16]
                for j in range(4):
                    acc[j] = acc[j] + e * wtmv[k, pl.ds(j * 16, 16)]
            for j in range(4):
                outb[r, pl.ds(j * 16, 16)] = acc[j]

        @pl.when(g == 2)
        def _():
            ev = [buft[r, pl.ds(h * 16, 16)] for h in range(DIM_TAIL // 16)]
            acc = [btv[pl.ds(j * 16, 16)] for j in range(4)]
            for k in range(DIM_TAIL):
                e = ev[k // 16][k % 16]
                for j in range(4):
                    acc[j] = acc[j] + e * wttv[k, pl.ds(j * 16, 16)]
            for j in range(4):
                outb[r, pl.ds(j * 16, 16)] = acc[j]

        return carry

    lax.fori_loop(0, RPW, row, 0)
    pltpu.sync_copy(outb, out_hbm.at[wid])


@jax.jit
def _run(xr, gr, head_table, mid_table, tail_table, wtm, b_mid, wtt, b_tail):
    mesh = plsc.VectorSubcoreMesh(core_axis_name="c", subcore_axis_name="s")
    f = functools.partial(
        pl.kernel,
        mesh=mesh,
        compiler_params=pltpu.CompilerParams(use_tc_tiling_on_sc=False),
        out_type=jax.ShapeDtypeStruct((NW, RPW, UNIFIED), jnp.float32),
        scratch_types=[
            pltpu.VMEM((NCH, GCH), jnp.int32),
            pltpu.VMEM((RPW + 16,), jnp.int32),
            pltpu.VMEM((RPW, UNIFIED), jnp.float32),
            pltpu.VMEM((RPW, DIM_MID), jnp.float32),
            pltpu.VMEM((RPW, DIM_TAIL), jnp.float32),
            pltpu.VMEM((RPW, UNIFIED), jnp.float32),
            pltpu.VMEM((DIM_MID, UNIFIED), jnp.float32),
            pltpu.VMEM((UNIFIED,), jnp.float32),
            pltpu.VMEM((DIM_TAIL, UNIFIED), jnp.float32),
            pltpu.VMEM((UNIFIED,), jnp.float32),
            pltpu.SemaphoreType.DMA,
        ],
    )(_body)
    return f(xr, gr, head_table, mid_table, tail_table, wtm, b_mid, wtt,
             b_tail)


def kernel(x, frequency_groups, head_table, mid_table, tail_table, W_mid,
           b_mid, W_tail, b_tail):
    xr = x.reshape(NW, NCH, GCH)
    gr = frequency_groups.reshape(NW, RPW)
    wtm = W_mid.T
    wtt = W_tail.T
    out = _run(xr, gr, head_table, mid_table, tail_table, wtm, b_mid, wtt,
               b_tail)
    return out.reshape(B, UNIFIED)
